# baseline (speedup 1.0000x reference)
import functools

import jax
import jax.numpy as jnp
from jax import lax
from jax.experimental import pallas as pl
from jax.experimental.pallas import tpu as pltpu
from jax.experimental.pallas import tpu_sc as plsc

N = 512
TH = 1.0
TAU = 0.1
NCH = 32
L = 16
NCHUNK = N // L
WORDS = N // 16


ROWS_PER_IT = 8


def _tc_body(x_ref, v_ref, w1b_ref, w2b_ref, b1_ref, scale_ref, beta_ref,
             b2_ref, vout_ref, ebits_ref, cmax_ref, a_ref, blk_ref):
    x = x_ref[...]
    v16 = v_ref[...]
    w1b = w1b_ref[...]
    w2b = w2b_ref[...]
    b1c = b1_ref[...]
    scalec = scale_ref[...]
    betac = beta_ref[...]
    b2 = b2_ref[0]

    col_iota = lax.broadcasted_iota(jnp.int32, (N, ROWS_PER_IT), 0)
    sub_iota = lax.broadcasted_iota(jnp.int32, (N, ROWS_PER_IT), 1)

    def row_block(k, carry):
        base = pl.multiple_of(k * ROWS_PER_IT, ROWS_PER_IT)
        onehot = (col_iota == sub_iota + base).astype(jnp.float32)
        x8 = jnp.dot(x, onehot, preferred_element_type=jnp.float32,
                     precision=lax.Precision.HIGHEST)
        for s in range(ROWS_PER_IT):
            xi = x8[:, s:s + 1]
            tb = (xi - x).astype(jnp.bfloat16)
            y = jnp.dot(w1b, tb,
                        preferred_element_type=jnp.float32)
            h = jnp.maximum(y + b1c, 0.0) * scalec + betac
            o2 = jnp.dot(w2b, h.astype(jnp.bfloat16),
                         preferred_element_type=jnp.float32)
            blk_ref[s:s + 1, :] = jnp.exp(o2 + b2)
        a_ref[pl.ds(base, ROWS_PER_IT), :] = blk_ref[...]
        return carry

    lax.fori_loop(0, N // ROWS_PER_IT, row_block, jnp.int32(0))

    e_half = a_ref[...]
    dm = 0.5 * (e_half + e_half.T)

    z = (TH - dm) * (1.0 / TAU)
    sig = 1.0 / (1.0 + jnp.exp(-z))
    colsum = jnp.sum(sig, axis=0, keepdims=True)
    sig_norm = sig / colsum
    v_soft = jnp.dot(v16, sig_norm, preferred_element_type=jnp.float32,
                     precision=lax.Precision.HIGHEST)
    vout_ref[...] = (v16 - v_soft) + v_soft

    ri = lax.broadcasted_iota(jnp.int32, (N, N), 0)
    ci = lax.broadcasted_iota(jnp.int32, (N, N), 1)
    e = (ci < ri) & (dm <= TH)
    cmax = jnp.max(jnp.where(e, ci, -1), axis=1, keepdims=True)
    cmax_ref[...] = cmax

    eprime = (e & (ci != cmax)).astype(jnp.float32)
    rc = lax.broadcasted_iota(jnp.int32, (N, WORDS), 0)
    wc = lax.broadcasted_iota(jnp.int32, (N, WORDS), 1)
    pmat = jnp.where((rc >> 4) == wc,
                     (jnp.int32(1) << (rc & 15)), 0).astype(jnp.float32)
    ebits_f = jnp.dot(eprime, pmat, preferred_element_type=jnp.float32,
                      precision=lax.Precision.HIGHEST)
    ebits_ref[...] = ebits_f.astype(jnp.int32)


def _run_tc(x, v16, w1b, w2b, b1c, scalec, betac, b2):
    return pl.pallas_call(
        _tc_body,
        out_shape=(
            jax.ShapeDtypeStruct((L, N), jnp.float32),
            jax.ShapeDtypeStruct((N, WORDS), jnp.int32),
            jax.ShapeDtypeStruct((N, 1), jnp.int32),
        ),
        in_specs=[
            pl.BlockSpec((L, N), lambda: (0, 0)),
            pl.BlockSpec((L, N), lambda: (0, 0)),
            pl.BlockSpec((NCH, L), lambda: (0, 0)),
            pl.BlockSpec((1, NCH), lambda: (0, 0)),
            pl.BlockSpec((NCH, 1), lambda: (0, 0)),
            pl.BlockSpec((NCH, 1), lambda: (0, 0)),
            pl.BlockSpec((NCH, 1), lambda: (0, 0)),
            pl.BlockSpec(memory_space=pltpu.SMEM),
        ],
        scratch_shapes=[pltpu.VMEM((N, N), jnp.float32),
                        pltpu.VMEM((ROWS_PER_IT, N), jnp.float32)],
    )(x, v16, w1b, w2b, b1c, scalec, betac, b2)


@functools.cache
def _sc_make():
    mesh = plsc.VectorSubcoreMesh(core_axis_name="c", subcore_axis_name="s")

    @functools.partial(
        pl.kernel, mesh=mesh,
        compiler_params=pltpu.CompilerParams(needs_layout_passes=False),
        out_type=jax.ShapeDtypeStruct((N,), jnp.int32),
        scratch_types=[
            pltpu.VMEM((N, WORDS), jnp.int32),
            pltpu.VMEM((N + L,), jnp.int32),
            pltpu.VMEM((N + L,), jnp.int32),
            pltpu.VMEM((N + L,), jnp.int32),
            pltpu.VMEM((N,), jnp.int32),
            pltpu.VMEM((N,), jnp.int32),
        ],
    )
    def sc_prog(ebits_hbm, cmax_hbm, out_hbm,
                ebits_v, cmax_v, labels_v, rowlist_v, rank_v, out_v):
        cid = lax.axis_index("c")
        sid = lax.axis_index("s")
        is_leader = jnp.logical_and(cid == 0, sid == 0)

        @pl.when(is_leader)
        def _():
            pltpu.sync_copy(ebits_hbm, ebits_v)
            pltpu.sync_copy(cmax_hbm, cmax_v.at[pl.ds(0, N)])

            lane = lax.iota(jnp.int32, L)

            def init_chunk(k, cnt):
                base = k * L
                basev = jnp.full((L,), base, dtype=jnp.int32)
                rows = lane + basev
                labels_v[pl.ds(base, L)] = rows
                av = cmax_v[pl.ds(base, L)] >= 0
                avi = av.astype(jnp.int32)
                pos = plsc.cumsum(avi) + jnp.full((L,), cnt - 1, jnp.int32)
                plsc.store_scatter(rowlist_v, [pos], rows, mask=av)
                return cnt + jnp.sum(avi)

            nrows = lax.fori_loop(0, NCHUNK, init_chunk, jnp.int32(0),
                                  unroll=False)

            def do_row(t, carry):
                t_vec = jnp.full((L,), t, dtype=jnp.int32)
                r_vec = plsc.load_gather(rowlist_v, [t_vec])
                cmax_vec = plsc.load_gather(cmax_v, [r_vec])
                l0_vec = plsc.load_gather(labels_v, [r_vec])

                def chunk(k, c2):
                    lab = labels_v[pl.ds(k * L, L)]
                    words = plsc.load_gather(ebits_v, [r_vec, lab >> 4])
                    bit = (words >> (lab & 15)) & 1
                    m = (bit != 0) | (lab == l0_vec)
                    labels_v[pl.ds(k * L, L)] = jnp.where(m, cmax_vec, lab)
                    return c2

                return lax.fori_loop(0, NCHUNK, chunk, carry, unroll=False)

            lax.fori_loop(0, nrows, do_row, jnp.int32(0), unroll=False)

            def zero_chunk(k, c):
                rank_v[pl.ds(k * L, L)] = jnp.zeros((L,), jnp.int32)
                return c
            lax.fori_loop(0, NCHUNK, zero_chunk, jnp.int32(0), unroll=False)

            ones = jnp.ones((L,), jnp.int32)

            def mark_chunk(k, c):
                lab = labels_v[pl.ds(k * L, L)]
                plsc.store_scatter(rank_v, [lab], ones)
                return c
            lax.fori_loop(0, NCHUNK, mark_chunk, jnp.int32(0), unroll=False)

            def rank_chunk(k, cnt):
                p = rank_v[pl.ds(k * L, L)]
                rank_v[pl.ds(k * L, L)] = (
                    plsc.cumsum(p) + jnp.full((L,), cnt - 1, jnp.int32))
                return cnt + jnp.sum(p)
            lax.fori_loop(0, NCHUNK, rank_chunk, jnp.int32(0), unroll=False)

            def out_chunk(k, c):
                lab = labels_v[pl.ds(k * L, L)]
                out_v[pl.ds(k * L, L)] = plsc.load_gather(rank_v, [lab])
                return c
            lax.fori_loop(0, NCHUNK, out_chunk, jnp.int32(0), unroll=False)

            pltpu.sync_copy(out_v, out_hbm)

    return sc_prog


def kernel(v, v_abs, W1, b1, gamma, beta, W2, b2):
    x = v_abs.reshape(L, N)
    v16 = v.reshape(L, N)
    w1b = W1[:, :, 0, 0].astype(jnp.bfloat16)
    w2b = W2[:, :, 0, 0].astype(jnp.bfloat16)
    scale = gamma / jnp.sqrt(1.0 + 1e-5)

    vout16, ebits, cmax2d = _run_tc(
        x, v16, w1b, w2b, b1.reshape(NCH, 1), scale.reshape(NCH, 1),
        beta.reshape(NCH, 1), b2)

    indices = _sc_make()(ebits, cmax2d.reshape(N))
    return (vout16.reshape(v.shape), indices)


# speedup vs baseline: 1.2975x; 1.2975x over previous
import functools

import jax
import jax.numpy as jnp
from jax import lax
from jax.experimental import pallas as pl
from jax.experimental.pallas import tpu as pltpu
from jax.experimental.pallas import tpu_sc as plsc

N = 512
TH = 1.0
TAU = 0.1
NCH = 32
L = 16
NCHUNK = N // L
WORDS = N // 16


ROWS_PER_IT = 8


def _tc_body(x_ref, v_ref, rmat_ref, m_ref, w1e_ref, w2e_ref,
             b1r_ref, scaler_ref, betar_ref, b2_ref,
             vout_ref, ebits_ref, cmax_ref, a_ref):
    B = ROWS_PER_IT
    x = x_ref[...]
    v16 = v_ref[...]
    rmat = rmat_ref[...]
    mmask = m_ref[...]
    w1e = w1e_ref[...]
    w2e = w2e_ref[...]
    b1r = b1r_ref[...]
    scaler = scaler_ref[...]
    betar = betar_ref[...]
    b2 = b2_ref[0]

    xrep = jnp.dot(rmat, x, preferred_element_type=jnp.float32,
                   precision=lax.Precision.HIGHEST)
    col_iota = lax.broadcasted_iota(jnp.int32, (N, B), 0)
    sub_iota = lax.broadcasted_iota(jnp.int32, (N, B), 1)

    def row_block(k, carry):
        base = pl.multiple_of(k * B, B)
        onehot = (col_iota == sub_iota + base).astype(jnp.float32)
        x8 = jnp.dot(x, onehot, preferred_element_type=jnp.float32,
                     precision=lax.Precision.HIGHEST)
        g = jnp.dot(rmat, x8, preferred_element_type=jnp.float32,
                    precision=lax.Precision.HIGHEST)
        xcol8 = jnp.sum(g * mmask, axis=1, keepdims=True)
        tb8 = (xcol8 - xrep).astype(jnp.bfloat16)
        y8 = jnp.dot(w1e, tb8,
                     preferred_element_type=jnp.float32)
        h8 = jnp.maximum(y8 + b1r, 0.0) * scaler + betar
        o28 = jnp.dot(w2e, h8.astype(jnp.bfloat16),
                      preferred_element_type=jnp.float32)
        a_ref[pl.ds(base, B), :] = jnp.exp(o28 + b2)
        return carry

    lax.fori_loop(0, N // ROWS_PER_IT, row_block, jnp.int32(0))

    e_half = a_ref[...]
    dm = 0.5 * (e_half + e_half.T)

    z = (TH - dm) * (1.0 / TAU)
    sig = 1.0 / (1.0 + jnp.exp(-z))
    colsum = jnp.sum(sig, axis=0, keepdims=True)
    sig_norm = sig / colsum
    v_soft = jnp.dot(v16, sig_norm, preferred_element_type=jnp.float32,
                     precision=lax.Precision.HIGHEST)
    vout_ref[...] = (v16 - v_soft) + v_soft

    ri = lax.broadcasted_iota(jnp.int32, (N, N), 0)
    ci = lax.broadcasted_iota(jnp.int32, (N, N), 1)
    e = (ci < ri) & (dm <= TH)
    cmax = jnp.max(jnp.where(e, ci, -1), axis=1, keepdims=True)
    cmax_ref[...] = cmax

    eprime = (e & (ci != cmax)).astype(jnp.float32)
    rc = lax.broadcasted_iota(jnp.int32, (N, WORDS), 0)
    wc = lax.broadcasted_iota(jnp.int32, (N, WORDS), 1)
    pmat = jnp.where((rc >> 4) == wc,
                     (jnp.int32(1) << (rc & 15)), 0).astype(jnp.float32)
    ebits_f = jnp.dot(eprime, pmat, preferred_element_type=jnp.float32,
                      precision=lax.Precision.HIGHEST)
    ebits_ref[...] = ebits_f.astype(jnp.int32)


def _run_tc(x, v16, rmat, mmask, w1e, w2e, b1r, scaler, betar, b2):
    B = ROWS_PER_IT
    return pl.pallas_call(
        _tc_body,
        out_shape=(
            jax.ShapeDtypeStruct((L, N), jnp.float32),
            jax.ShapeDtypeStruct((N, WORDS), jnp.int32),
            jax.ShapeDtypeStruct((N, 1), jnp.int32),
        ),
        in_specs=[
            pl.BlockSpec((L, N), lambda: (0, 0)),
            pl.BlockSpec((L, N), lambda: (0, 0)),
            pl.BlockSpec((B * L, L), lambda: (0, 0)),
            pl.BlockSpec((B * L, B), lambda: (0, 0)),
            pl.BlockSpec((B * NCH, B * L), lambda: (0, 0)),
            pl.BlockSpec((B, B * NCH), lambda: (0, 0)),
            pl.BlockSpec((B * NCH, 1), lambda: (0, 0)),
            pl.BlockSpec((B * NCH, 1), lambda: (0, 0)),
            pl.BlockSpec((B * NCH, 1), lambda: (0, 0)),
            pl.BlockSpec(memory_space=pltpu.SMEM),
        ],
        scratch_shapes=[pltpu.VMEM((N, N), jnp.float32)],
    )(x, v16, rmat, mmask, w1e, w2e, b1r, scaler, betar, b2)


@functools.cache
def _sc_make():
    mesh = plsc.VectorSubcoreMesh(core_axis_name="c", subcore_axis_name="s")

    @functools.partial(
        pl.kernel, mesh=mesh,
        compiler_params=pltpu.CompilerParams(needs_layout_passes=False),
        out_type=jax.ShapeDtypeStruct((N,), jnp.int32),
        scratch_types=[
            pltpu.VMEM((N, WORDS), jnp.int32),
            pltpu.VMEM((N + L,), jnp.int32),
            pltpu.VMEM((N + L,), jnp.int32),
            pltpu.VMEM((N + L,), jnp.int32),
            pltpu.VMEM((N,), jnp.int32),
            pltpu.VMEM((N,), jnp.int32),
        ],
    )
    def sc_prog(ebits_hbm, cmax_hbm, out_hbm,
                ebits_v, cmax_v, labels_v, rowlist_v, rank_v, out_v):
        cid = lax.axis_index("c")
        sid = lax.axis_index("s")
        is_leader = jnp.logical_and(cid == 0, sid == 0)

        @pl.when(is_leader)
        def _():
            pltpu.sync_copy(ebits_hbm, ebits_v)
            pltpu.sync_copy(cmax_hbm, cmax_v.at[pl.ds(0, N)])

            lane = lax.iota(jnp.int32, L)

            def init_chunk(k, cnt):
                base = k * L
                basev = jnp.full((L,), base, dtype=jnp.int32)
                rows = lane + basev
                labels_v[pl.ds(base, L)] = rows
                av = cmax_v[pl.ds(base, L)] >= 0
                avi = av.astype(jnp.int32)
                pos = plsc.cumsum(avi) + jnp.full((L,), cnt - 1, jnp.int32)
                plsc.store_scatter(rowlist_v, [pos], rows, mask=av)
                return cnt + jnp.sum(avi)

            nrows = lax.fori_loop(0, NCHUNK, init_chunk, jnp.int32(0),
                                  unroll=False)

            def do_row(t, carry):
                t_vec = jnp.full((L,), t, dtype=jnp.int32)
                r_vec = plsc.load_gather(rowlist_v, [t_vec])
                cmax_vec = plsc.load_gather(cmax_v, [r_vec])
                l0_vec = plsc.load_gather(labels_v, [r_vec])

                def chunk(k, c2):
                    lab = labels_v[pl.ds(k * L, L)]
                    words = plsc.load_gather(ebits_v, [r_vec, lab >> 4])
                    bit = (words >> (lab & 15)) & 1
                    m = (bit != 0) | (lab == l0_vec)
                    labels_v[pl.ds(k * L, L)] = jnp.where(m, cmax_vec, lab)
                    return c2

                return lax.fori_loop(0, NCHUNK, chunk, carry, unroll=False)

            lax.fori_loop(0, nrows, do_row, jnp.int32(0), unroll=False)

            def zero_chunk(k, c):
                rank_v[pl.ds(k * L, L)] = jnp.zeros((L,), jnp.int32)
                return c
            lax.fori_loop(0, NCHUNK, zero_chunk, jnp.int32(0), unroll=False)

            ones = jnp.ones((L,), jnp.int32)

            def mark_chunk(k, c):
                lab = labels_v[pl.ds(k * L, L)]
                plsc.store_scatter(rank_v, [lab], ones)
                return c
            lax.fori_loop(0, NCHUNK, mark_chunk, jnp.int32(0), unroll=False)

            def rank_chunk(k, cnt):
                p = rank_v[pl.ds(k * L, L)]
                rank_v[pl.ds(k * L, L)] = (
                    plsc.cumsum(p) + jnp.full((L,), cnt - 1, jnp.int32))
                return cnt + jnp.sum(p)
            lax.fori_loop(0, NCHUNK, rank_chunk, jnp.int32(0), unroll=False)

            def out_chunk(k, c):
                lab = labels_v[pl.ds(k * L, L)]
                out_v[pl.ds(k * L, L)] = plsc.load_gather(rank_v, [lab])
                return c
            lax.fori_loop(0, NCHUNK, out_chunk, jnp.int32(0), unroll=False)

            pltpu.sync_copy(out_v, out_hbm)

    return sc_prog


def kernel(v, v_abs, W1, b1, gamma, beta, W2, b2):
    B = ROWS_PER_IT
    x = v_abs.reshape(L, N)
    v16 = v.reshape(L, N)
    w1f = W1[:, :, 0, 0].astype(jnp.bfloat16).astype(jnp.float32)
    w2f = W2[:, :, 0, 0].astype(jnp.bfloat16).astype(jnp.float32)
    eyeb = jnp.eye(B, dtype=jnp.float32)
    w1e = jnp.kron(eyeb, w1f).astype(jnp.bfloat16)
    w2e = jnp.kron(eyeb, w2f).astype(jnp.bfloat16)
    rmat = jnp.tile(jnp.eye(L, dtype=jnp.float32), (B, 1))
    mmask = jnp.kron(eyeb, jnp.ones((L, 1), jnp.float32))
    scale = gamma / jnp.sqrt(1.0 + 1e-5)
    b1r = jnp.tile(b1, B).reshape(B * NCH, 1)
    scaler = jnp.tile(scale, B).reshape(B * NCH, 1)
    betar = jnp.tile(beta, B).reshape(B * NCH, 1)

    vout16, ebits, cmax2d = _run_tc(
        x, v16, rmat, mmask, w1e, w2e, b1r, scaler, betar, b2)

    indices = _sc_make()(ebits, cmax2d.reshape(N))
    return (vout16.reshape(v.shape), indices)


# speedup vs baseline: 1.6619x; 1.2809x over previous
import functools

import jax
import jax.numpy as jnp
from jax import lax
from jax.experimental import pallas as pl
from jax.experimental.pallas import tpu as pltpu
from jax.experimental.pallas import tpu_sc as plsc

N = 512
TH = 1.0
TAU = 0.1
NCH = 32
L = 16
NCHUNK = N // L
WORDS = N // 16


ROWS_PER_IT = 16


def _tc_body(x_ref, v_ref, rmat_ref, m_ref, w1e_ref, w2e_ref,
             b1r_ref, scaler_ref, betar_ref, b2_ref,
             vout_ref, ebits_ref, cmax_ref, a_ref):
    B = ROWS_PER_IT
    x = x_ref[...]
    v16 = v_ref[...]
    rmat = rmat_ref[...]
    mmask = m_ref[...]
    w1e = w1e_ref[...]
    w2e = w2e_ref[...]
    b1r = b1r_ref[...]
    scaler = scaler_ref[...]
    betar = betar_ref[...]
    b2 = b2_ref[0]

    xrep = jnp.dot(rmat, x, preferred_element_type=jnp.float32,
                   precision=lax.Precision.HIGHEST)
    col_iota = lax.broadcasted_iota(jnp.int32, (N, B), 0)
    sub_iota = lax.broadcasted_iota(jnp.int32, (N, B), 1)

    def row_block(k, carry):
        base = pl.multiple_of(k * B, B)
        onehot = (col_iota == sub_iota + base).astype(jnp.float32)
        x8 = jnp.dot(x, onehot, preferred_element_type=jnp.float32,
                     precision=lax.Precision.HIGHEST)
        g = jnp.dot(rmat, x8, preferred_element_type=jnp.float32,
                    precision=lax.Precision.HIGHEST)
        xcol8 = jnp.sum(g * mmask, axis=1, keepdims=True)
        tb8 = (xcol8 - xrep).astype(jnp.bfloat16)
        y8 = jnp.dot(w1e, tb8,
                     preferred_element_type=jnp.float32)
        h8 = jnp.maximum(y8 + b1r, 0.0) * scaler + betar
        o28 = jnp.dot(w2e, h8.astype(jnp.bfloat16),
                      preferred_element_type=jnp.float32)
        a_ref[pl.ds(base, B), :] = jnp.exp(o28 + b2)
        return carry

    lax.fori_loop(0, N // ROWS_PER_IT, row_block, jnp.int32(0), unroll=2)

    e_half = a_ref[...]
    dm = 0.5 * (e_half + e_half.T)

    z = (TH - dm) * (1.0 / TAU)
    sig = 1.0 / (1.0 + jnp.exp(-z))
    colsum = jnp.sum(sig, axis=0, keepdims=True)
    sig_norm = sig / colsum
    v_soft = jnp.dot(v16, sig_norm, preferred_element_type=jnp.float32,
                     precision=lax.Precision.HIGHEST)
    vout_ref[...] = (v16 - v_soft) + v_soft

    ri = lax.broadcasted_iota(jnp.int32, (N, N), 0)
    ci = lax.broadcasted_iota(jnp.int32, (N, N), 1)
    e = (ci < ri) & (dm <= TH)
    cmax = jnp.max(jnp.where(e, ci, -1), axis=1, keepdims=True)
    cmax_ref[...] = cmax

    eprime = (e & (ci != cmax)).astype(jnp.float32)
    rc = lax.broadcasted_iota(jnp.int32, (N, WORDS), 0)
    wc = lax.broadcasted_iota(jnp.int32, (N, WORDS), 1)
    pmat = jnp.where((rc >> 4) == wc,
                     (jnp.int32(1) << (rc & 15)), 0).astype(jnp.float32)
    ebits_f = jnp.dot(eprime, pmat, preferred_element_type=jnp.float32,
                      precision=lax.Precision.HIGHEST)
    ebits_ref[...] = ebits_f.astype(jnp.int32)


def _run_tc(x, v16, rmat, mmask, w1e, w2e, b1r, scaler, betar, b2):
    B = ROWS_PER_IT
    return pl.pallas_call(
        _tc_body,
        out_shape=(
            jax.ShapeDtypeStruct((L, N), jnp.float32),
            jax.ShapeDtypeStruct((N, WORDS), jnp.int32),
            jax.ShapeDtypeStruct((N, 1), jnp.int32),
        ),
        in_specs=[
            pl.BlockSpec((L, N), lambda: (0, 0)),
            pl.BlockSpec((L, N), lambda: (0, 0)),
            pl.BlockSpec((B * L, L), lambda: (0, 0)),
            pl.BlockSpec((B * L, B), lambda: (0, 0)),
            pl.BlockSpec((B * NCH, B * L), lambda: (0, 0)),
            pl.BlockSpec((B, B * NCH), lambda: (0, 0)),
            pl.BlockSpec((B * NCH, 1), lambda: (0, 0)),
            pl.BlockSpec((B * NCH, 1), lambda: (0, 0)),
            pl.BlockSpec((B * NCH, 1), lambda: (0, 0)),
            pl.BlockSpec(memory_space=pltpu.SMEM),
        ],
        scratch_shapes=[pltpu.VMEM((N, N), jnp.float32)],
    )(x, v16, rmat, mmask, w1e, w2e, b1r, scaler, betar, b2)


@functools.cache
def _sc_make():
    mesh = plsc.VectorSubcoreMesh(core_axis_name="c", subcore_axis_name="s")

    @functools.partial(
        pl.kernel, mesh=mesh,
        compiler_params=pltpu.CompilerParams(needs_layout_passes=False),
        out_type=jax.ShapeDtypeStruct((N,), jnp.int32),
        scratch_types=[
            pltpu.VMEM((N, WORDS), jnp.int32),
            pltpu.VMEM((N + L,), jnp.int32),
            pltpu.VMEM((N + L,), jnp.int32),
            pltpu.VMEM((N + L,), jnp.int32),
            pltpu.VMEM((N,), jnp.int32),
            pltpu.VMEM((N,), jnp.int32),
        ],
    )
    def sc_prog(ebits_hbm, cmax_hbm, out_hbm,
                ebits_v, cmax_v, labels_v, rowlist_v, rank_v, out_v):
        cid = lax.axis_index("c")
        sid = lax.axis_index("s")
        is_leader = jnp.logical_and(cid == 0, sid == 0)

        @pl.when(is_leader)
        def _():
            pltpu.sync_copy(ebits_hbm, ebits_v)
            pltpu.sync_copy(cmax_hbm, cmax_v.at[pl.ds(0, N)])

            lane = lax.iota(jnp.int32, L)

            def init_chunk(k, cnt):
                base = k * L
                basev = jnp.full((L,), base, dtype=jnp.int32)
                rows = lane + basev
                labels_v[pl.ds(base, L)] = rows
                av = cmax_v[pl.ds(base, L)] >= 0
                avi = av.astype(jnp.int32)
                pos = plsc.cumsum(avi) + jnp.full((L,), cnt - 1, jnp.int32)
                plsc.store_scatter(rowlist_v, [pos], rows, mask=av)
                return cnt + jnp.sum(avi)

            nrows = lax.fori_loop(0, NCHUNK, init_chunk, jnp.int32(0),
                                  unroll=False)

            def do_row(t, carry):
                t_vec = jnp.full((L,), t, dtype=jnp.int32)
                r_vec = plsc.load_gather(rowlist_v, [t_vec])
                cmax_vec = plsc.load_gather(cmax_v, [r_vec])
                l0_vec = plsc.load_gather(labels_v, [r_vec])

                def chunk(k, c2):
                    lab = labels_v[pl.ds(k * L, L)]
                    words = plsc.load_gather(ebits_v, [r_vec, lab >> 4])
                    bit = (words >> (lab & 15)) & 1
                    m = (bit != 0) | (lab == l0_vec)
                    labels_v[pl.ds(k * L, L)] = jnp.where(m, cmax_vec, lab)
                    return c2

                return lax.fori_loop(0, NCHUNK, chunk, carry, unroll=False)

            lax.fori_loop(0, nrows, do_row, jnp.int32(0), unroll=False)

            def zero_chunk(k, c):
                rank_v[pl.ds(k * L, L)] = jnp.zeros((L,), jnp.int32)
                return c
            lax.fori_loop(0, NCHUNK, zero_chunk, jnp.int32(0), unroll=False)

            ones = jnp.ones((L,), jnp.int32)

            def mark_chunk(k, c):
                lab = labels_v[pl.ds(k * L, L)]
                plsc.store_scatter(rank_v, [lab], ones)
                return c
            lax.fori_loop(0, NCHUNK, mark_chunk, jnp.int32(0), unroll=False)

            def rank_chunk(k, cnt):
                p = rank_v[pl.ds(k * L, L)]
                rank_v[pl.ds(k * L, L)] = (
                    plsc.cumsum(p) + jnp.full((L,), cnt - 1, jnp.int32))
                return cnt + jnp.sum(p)
            lax.fori_loop(0, NCHUNK, rank_chunk, jnp.int32(0), unroll=False)

            def out_chunk(k, c):
                lab = labels_v[pl.ds(k * L, L)]
                out_v[pl.ds(k * L, L)] = plsc.load_gather(rank_v, [lab])
                return c
            lax.fori_loop(0, NCHUNK, out_chunk, jnp.int32(0), unroll=False)

            pltpu.sync_copy(out_v, out_hbm)

    return sc_prog


def kernel(v, v_abs, W1, b1, gamma, beta, W2, b2):
    B = ROWS_PER_IT
    x = v_abs.reshape(L, N)
    v16 = v.reshape(L, N)
    w1f = W1[:, :, 0, 0].astype(jnp.bfloat16).astype(jnp.float32)
    w2f = W2[:, :, 0, 0].astype(jnp.bfloat16).astype(jnp.float32)
    eyeb = jnp.eye(B, dtype=jnp.float32)
    w1e = jnp.kron(eyeb, w1f).astype(jnp.bfloat16)
    w2e = jnp.kron(eyeb, w2f).astype(jnp.bfloat16)
    rmat = jnp.tile(jnp.eye(L, dtype=jnp.float32), (B, 1))
    mmask = jnp.kron(eyeb, jnp.ones((L, 1), jnp.float32))
    scale = gamma / jnp.sqrt(1.0 + 1e-5)
    b1r = jnp.tile(b1, B).reshape(B * NCH, 1)
    scaler = jnp.tile(scale, B).reshape(B * NCH, 1)
    betar = jnp.tile(beta, B).reshape(B * NCH, 1)

    vout16, ebits, cmax2d = _run_tc(
        x, v16, rmat, mmask, w1e, w2e, b1r, scaler, betar, b2)

    indices = _sc_make()(ebits, cmax2d.reshape(N))
    return (vout16.reshape(v.shape), indices)


# speedup vs baseline: 1.8526x; 1.1148x over previous
import functools

import jax
import jax.numpy as jnp
from jax import lax
from jax.experimental import pallas as pl
from jax.experimental.pallas import tpu as pltpu
from jax.experimental.pallas import tpu_sc as plsc

N = 512
TH = 1.0
TAU = 0.1
NCH = 32
L = 16
NCHUNK = N // L
WORDS = N // 16


ROWS_PER_IT = 16


def _tc_body(x_ref, v_ref, rmat_ref, m_ref, w1e_ref, w2e_ref,
             b1r_ref, scaler_ref, betar_ref, b2_ref,
             vout_ref, ebits_ref, cmax_ref, a_ref):
    B = ROWS_PER_IT
    x = x_ref[...]
    v16 = v_ref[...]
    rmat = rmat_ref[...]
    mmask = m_ref[...]
    w1e = w1e_ref[...]
    w2e = w2e_ref[...]
    b1r = b1r_ref[...]
    scaler = scaler_ref[...]
    betar = betar_ref[...]
    b2 = b2_ref[0]

    xrep = jnp.dot(rmat, x, preferred_element_type=jnp.float32,
                   precision=lax.Precision.HIGHEST)
    col_iota = lax.broadcasted_iota(jnp.int32, (N, B), 0)
    sub_iota = lax.broadcasted_iota(jnp.int32, (N, B), 1)

    def row_block(k, carry):
        base = pl.multiple_of(k * B, B)
        onehot = (col_iota == sub_iota + base).astype(jnp.float32)
        x8 = jnp.dot(x, onehot, preferred_element_type=jnp.float32,
                     precision=lax.Precision.HIGHEST)
        g = jnp.dot(rmat, x8, preferred_element_type=jnp.float32,
                    precision=lax.Precision.HIGHEST)
        xcol8 = jnp.sum(g * mmask, axis=1, keepdims=True)
        tb8 = (xcol8 - xrep).astype(jnp.bfloat16)
        y8 = jnp.dot(w1e, tb8,
                     preferred_element_type=jnp.float32)
        h8 = jnp.maximum(y8 + b1r, 0.0) * scaler + betar
        o28 = jnp.dot(w2e, h8.astype(jnp.bfloat16),
                      preferred_element_type=jnp.float32)
        a_ref[pl.ds(base, B), :] = jnp.exp(o28 + b2)
        return carry

    lax.fori_loop(0, N // ROWS_PER_IT, row_block, jnp.int32(0), unroll=4)

    e_half = a_ref[...]
    dm = 0.5 * (e_half + e_half.T)

    z = (TH - dm) * (1.0 / TAU)
    sig = 1.0 / (1.0 + jnp.exp(-z))
    colsum = jnp.sum(sig, axis=0, keepdims=True)
    sig_norm = sig / colsum
    v_soft = jnp.dot(v16, sig_norm, preferred_element_type=jnp.float32,
                     precision=lax.Precision.HIGHEST)
    vout_ref[...] = (v16 - v_soft) + v_soft

    ri = lax.broadcasted_iota(jnp.int32, (N, N), 0)
    ci = lax.broadcasted_iota(jnp.int32, (N, N), 1)
    e = (ci < ri) & (dm <= TH)
    cmax = jnp.max(jnp.where(e, ci, -1), axis=1, keepdims=True)
    cmax_ref[...] = cmax

    eprime = (e & (ci != cmax)).astype(jnp.float32)
    rc = lax.broadcasted_iota(jnp.int32, (N, WORDS), 0)
    wc = lax.broadcasted_iota(jnp.int32, (N, WORDS), 1)
    pmat = jnp.where((rc >> 4) == wc,
                     (jnp.int32(1) << (rc & 15)), 0).astype(jnp.float32)
    ebits_f = jnp.dot(eprime, pmat, preferred_element_type=jnp.float32,
                      precision=lax.Precision.HIGHEST)
    ebits_ref[...] = ebits_f.astype(jnp.int32)


def _run_tc(x, v16, rmat, mmask, w1e, w2e, b1r, scaler, betar, b2):
    B = ROWS_PER_IT
    return pl.pallas_call(
        _tc_body,
        out_shape=(
            jax.ShapeDtypeStruct((L, N), jnp.float32),
            jax.ShapeDtypeStruct((N, WORDS), jnp.int32),
            jax.ShapeDtypeStruct((N, 1), jnp.int32),
        ),
        in_specs=[
            pl.BlockSpec((L, N), lambda: (0, 0)),
            pl.BlockSpec((L, N), lambda: (0, 0)),
            pl.BlockSpec((B * L, L), lambda: (0, 0)),
            pl.BlockSpec((B * L, B), lambda: (0, 0)),
            pl.BlockSpec((B * NCH, B * L), lambda: (0, 0)),
            pl.BlockSpec((B, B * NCH), lambda: (0, 0)),
            pl.BlockSpec((B * NCH, 1), lambda: (0, 0)),
            pl.BlockSpec((B * NCH, 1), lambda: (0, 0)),
            pl.BlockSpec((B * NCH, 1), lambda: (0, 0)),
            pl.BlockSpec(memory_space=pltpu.SMEM),
        ],
        scratch_shapes=[pltpu.VMEM((N, N), jnp.float32)],
    )(x, v16, rmat, mmask, w1e, w2e, b1r, scaler, betar, b2)


@functools.cache
def _sc_make():
    mesh = plsc.VectorSubcoreMesh(core_axis_name="c", subcore_axis_name="s")

    @functools.partial(
        pl.kernel, mesh=mesh,
        compiler_params=pltpu.CompilerParams(needs_layout_passes=False),
        out_type=jax.ShapeDtypeStruct((N,), jnp.int32),
        scratch_types=[
            pltpu.VMEM((N, WORDS), jnp.int32),
            pltpu.VMEM((N + L,), jnp.int32),
            pltpu.VMEM((N + L,), jnp.int32),
            pltpu.VMEM((N + L,), jnp.int32),
            pltpu.VMEM((N,), jnp.int32),
            pltpu.VMEM((N,), jnp.int32),
        ],
    )
    def sc_prog(ebits_hbm, cmax_hbm, out_hbm,
                ebits_v, cmax_v, labels_v, rowlist_v, rank_v, out_v):
        cid = lax.axis_index("c")
        sid = lax.axis_index("s")
        is_leader = jnp.logical_and(cid == 0, sid == 0)

        @pl.when(is_leader)
        def _():
            pltpu.sync_copy(ebits_hbm, ebits_v)
            pltpu.sync_copy(cmax_hbm, cmax_v.at[pl.ds(0, N)])

            lane = lax.iota(jnp.int32, L)

            def init_chunk(k, cnt):
                base = k * L
                basev = jnp.full((L,), base, dtype=jnp.int32)
                rows = lane + basev
                labels_v[pl.ds(base, L)] = rows
                av = cmax_v[pl.ds(base, L)] >= 0
                avi = av.astype(jnp.int32)
                pos = plsc.cumsum(avi) + jnp.full((L,), cnt - 1, jnp.int32)
                plsc.store_scatter(rowlist_v, [pos], rows, mask=av)
                return cnt + jnp.sum(avi)

            nrows = lax.fori_loop(0, NCHUNK, init_chunk, jnp.int32(0),
                                  unroll=False)

            def do_row(t, carry):
                t_vec = jnp.full((L,), t, dtype=jnp.int32)
                r_vec = plsc.load_gather(rowlist_v, [t_vec])
                cmax_vec = plsc.load_gather(cmax_v, [r_vec])
                l0_vec = plsc.load_gather(labels_v, [r_vec])

                def chunk(k, c2):
                    lab = labels_v[pl.ds(k * L, L)]
                    words = plsc.load_gather(ebits_v, [r_vec, lab >> 4])
                    bit = (words >> (lab & 15)) & 1
                    m = (bit != 0) | (lab == l0_vec)
                    labels_v[pl.ds(k * L, L)] = jnp.where(m, cmax_vec, lab)
                    return c2

                return lax.fori_loop(0, NCHUNK, chunk, carry, unroll=False)

            lax.fori_loop(0, nrows, do_row, jnp.int32(0), unroll=False)

            def zero_chunk(k, c):
                rank_v[pl.ds(k * L, L)] = jnp.zeros((L,), jnp.int32)
                return c
            lax.fori_loop(0, NCHUNK, zero_chunk, jnp.int32(0), unroll=False)

            ones = jnp.ones((L,), jnp.int32)

            def mark_chunk(k, c):
                lab = labels_v[pl.ds(k * L, L)]
                plsc.store_scatter(rank_v, [lab], ones)
                return c
            lax.fori_loop(0, NCHUNK, mark_chunk, jnp.int32(0), unroll=False)

            def rank_chunk(k, cnt):
                p = rank_v[pl.ds(k * L, L)]
                rank_v[pl.ds(k * L, L)] = (
                    plsc.cumsum(p) + jnp.full((L,), cnt - 1, jnp.int32))
                return cnt + jnp.sum(p)
            lax.fori_loop(0, NCHUNK, rank_chunk, jnp.int32(0), unroll=False)

            def out_chunk(k, c):
                lab = labels_v[pl.ds(k * L, L)]
                out_v[pl.ds(k * L, L)] = plsc.load_gather(rank_v, [lab])
                return c
            lax.fori_loop(0, NCHUNK, out_chunk, jnp.int32(0), unroll=False)

            pltpu.sync_copy(out_v, out_hbm)

    return sc_prog


def kernel(v, v_abs, W1, b1, gamma, beta, W2, b2):
    B = ROWS_PER_IT
    x = v_abs.reshape(L, N)
    v16 = v.reshape(L, N)
    w1f = W1[:, :, 0, 0].astype(jnp.bfloat16).astype(jnp.float32)
    w2f = W2[:, :, 0, 0].astype(jnp.bfloat16).astype(jnp.float32)
    eyeb = jnp.eye(B, dtype=jnp.float32)
    w1e = jnp.kron(eyeb, w1f).astype(jnp.bfloat16)
    w2e = jnp.kron(eyeb, w2f).astype(jnp.bfloat16)
    rmat = jnp.tile(jnp.eye(L, dtype=jnp.float32), (B, 1))
    mmask = jnp.kron(eyeb, jnp.ones((L, 1), jnp.float32))
    scale = gamma / jnp.sqrt(1.0 + 1e-5)
    b1r = jnp.tile(b1, B).reshape(B * NCH, 1)
    scaler = jnp.tile(scale, B).reshape(B * NCH, 1)
    betar = jnp.tile(beta, B).reshape(B * NCH, 1)

    vout16, ebits, cmax2d = _run_tc(
        x, v16, rmat, mmask, w1e, w2e, b1r, scaler, betar, b2)

    indices = _sc_make()(ebits, cmax2d.reshape(N))
    return (vout16.reshape(v.shape), indices)
